# baseline (speedup 1.0000x reference)
import jax
import jax.numpy as jnp
from jax.experimental import pallas as pl

_BS = 512


def _pe_add_kernel(x_ref, pe_ref, out_ref):
    x = x_ref[...]
    pe = pe_ref[...]
    out_ref[...] = jnp.where(x == 0.0, x, x + pe[None, :, :])


def kernel(x, pos_embed):
    batch, seq, dim = x.shape
    pe = pos_embed[:seq]
    grid = (seq // _BS,)
    return pl.pallas_call(
        _pe_add_kernel,
        grid=grid,
        in_specs=[
            pl.BlockSpec((batch, _BS, dim), lambda s: (0, s, 0)),
            pl.BlockSpec((_BS, dim), lambda s: (s, 0)),
        ],
        out_specs=pl.BlockSpec((batch, _BS, dim), lambda s: (0, s, 0)),
        out_shape=jax.ShapeDtypeStruct(x.shape, x.dtype),
    )(x, pe)
